# baseline (speedup 1.0000x reference)
import dataclasses
import functools

import jax
import jax.numpy as jnp
from jax import lax
from jax.experimental import pallas as pl
from jax.experimental.pallas import tpu as pltpu
from jax.experimental.pallas import tpu_sc as plsc

EMBED_DIM = 64
SCALE = 8.0
VOCAB = 1000000
B = 4096
S = 200
BT = B // 128


def _compiler_params():
    cp = pltpu.CompilerParams(use_tc_tiling_on_sc=False)
    if "needs_layout_passes" in pltpu.CompilerParams.__dataclass_fields__:
        cp = dataclasses.replace(cp, needs_layout_passes=False)
    return cp


def kernel(x, table):
    tp = jnp.pad(table, ((0, 0), (0, 128 - EMBED_DIM)))
    tp2 = tp.reshape(2 * VOCAB, EMBED_DIM)
    xt2 = x.astype(jnp.int32).T * 2

    mesh = plsc.VectorSubcoreMesh(
        core_axis_name="core", subcore_axis_name="subcore"
    )

    @functools.partial(
        pl.kernel,
        out_type=jax.ShapeDtypeStruct((S, 8, BT, 8, 128), jnp.float32),
        mesh=mesh,
        compiler_params=_compiler_params(),
        scratch_types=[
            pltpu.VMEM((S, 128), jnp.int32),
            pltpu.VMEM((2, 128, EMBED_DIM), jnp.float32),
            pltpu.VMEM((2, EMBED_DIM, 128), jnp.float32),
            pltpu.SemaphoreType.DMA,
            pltpu.SemaphoreType.DMA,
            pltpu.SemaphoreType.DMA,
            pltpu.SemaphoreType.DMA,
        ],
    )
    def emb(table_hbm, idx_hbm, o_hbm, idx_v, rows_v, dst_v,
            sem_g0, sem_g1, sem_o0, sem_o1):
        wid = lax.axis_index("subcore") * 2 + lax.axis_index("core")
        lane = lax.iota(jnp.int32, 16)
        rvecs = [lane + (16 * lg) for lg in range(8)]
        sem_g = (sem_g0, sem_g1)
        sem_o = (sem_o0, sem_o1)
        rows = (rows_v.at[0], rows_v.at[1])
        dst = (dst_v.at[0], dst_v.at[1])

        pltpu.sync_copy(idx_hbm.at[:, pl.ds(wid * 128, 128)], idx_v)

        def gather(s, b):
            return pltpu.make_async_copy(
                table_hbm.at[idx_v.at[s]], rows[b], sem_g[b]
            )

        def out_copies(s, b):
            return [
                pltpu.make_async_copy(
                    dst[b].at[pl.ds(db * 8, 8)], o_hbm.at[s, db, wid],
                    sem_o[b],
                )
                for db in range(8)
            ]

        gather(0, 0).start()
        gather(1, 1).start()

        @pl.loop(0, S, step=2)
        def _(g):
            for b in range(2):
                s = g + b
                gather(s, b).wait()

                @pl.when(s >= 2)
                def _():
                    for c in out_copies(s - 2, b):
                        c.wait()

                @plsc.parallel_loop(0, EMBED_DIM, unroll=8)
                def _(d):
                    col = jnp.full((16,), d, dtype=jnp.int32)
                    for lg in range(8):
                        vals = plsc.load_gather(rows[b], [rvecs[lg], col])
                        dst[b][d, pl.ds(lg * 16, 16)] = vals * SCALE

                @pl.when(s < S - 2)
                def _():
                    gather(s + 2, b).start()

                for c in out_copies(s, b):
                    c.start()

        for b in range(2):
            for c in out_copies(S - 2 + b, b):
                c.wait()

    o5 = emb(tp2, xt2)
    return o5.transpose(2, 4, 0, 1, 3).reshape(B, S, EMBED_DIM)


# speedup vs baseline: 1.7671x; 1.7671x over previous
import dataclasses
import functools

import jax
import jax.numpy as jnp
from jax import lax
from jax.experimental import pallas as pl
from jax.experimental.pallas import tpu as pltpu
from jax.experimental.pallas import tpu_sc as plsc

EMBED_DIM = 64
SCALE = 8.0
VOCAB = 1000000
B = 4096
S = 200
BT = B // 128


def _compiler_params():
    cp = pltpu.CompilerParams(use_tc_tiling_on_sc=False)
    if "needs_layout_passes" in pltpu.CompilerParams.__dataclass_fields__:
        cp = dataclasses.replace(cp, needs_layout_passes=False)
    return cp


def kernel(x, table):
    tp = jnp.pad(table, ((0, 0), (0, 128 - EMBED_DIM)))
    tp2 = tp.reshape(2 * VOCAB, EMBED_DIM)
    xt2 = x.astype(jnp.int32).T * 2

    mesh = plsc.VectorSubcoreMesh(
        core_axis_name="core", subcore_axis_name="subcore"
    )

    @functools.partial(
        pl.kernel,
        out_type=jax.ShapeDtypeStruct((S, 8, BT, 8, 128), jnp.float32),
        mesh=mesh,
        compiler_params=_compiler_params(),
        scratch_types=[
            pltpu.VMEM((S, 128), jnp.int32),
            pltpu.VMEM((2, 128, EMBED_DIM), jnp.float32),
            pltpu.VMEM((2, EMBED_DIM, 129), jnp.float32),
            pltpu.SemaphoreType.DMA,
            pltpu.SemaphoreType.DMA,
            pltpu.SemaphoreType.DMA,
            pltpu.SemaphoreType.DMA,
        ],
    )
    def emb(table_hbm, idx_hbm, o_hbm, idx_v, rows_v, dst_v,
            sem_g0, sem_g1, sem_o0, sem_o1):
        wid = lax.axis_index("subcore") * 2 + lax.axis_index("core")
        lane = lax.iota(jnp.int32, 16)
        dvecs = [lane + (16 * dq) for dq in range(4)]
        sem_g = (sem_g0, sem_g1)
        sem_o = (sem_o0, sem_o1)
        rows = (rows_v.at[0], rows_v.at[1])
        dst = (dst_v.at[0], dst_v.at[1])

        pltpu.sync_copy(idx_hbm.at[:, pl.ds(wid * 128, 128)], idx_v)

        def gather(s, b):
            return pltpu.make_async_copy(
                table_hbm.at[idx_v.at[s]], rows[b], sem_g[b]
            )

        def out_copies(s, b):
            return [
                pltpu.make_async_copy(
                    dst[b].at[pl.ds(db * 8, 8), pl.ds(0, 128)],
                    o_hbm.at[s, db, wid],
                    sem_o[b],
                )
                for db in range(8)
            ]

        gather(0, 0).start()
        gather(1, 1).start()

        @pl.loop(0, S, step=2)
        def _(g):
            for b in range(2):
                s = g + b
                gather(s, b).wait()

                @pl.when(s >= 2)
                def _():
                    for c in out_copies(s - 2, b):
                        c.wait()

                @plsc.parallel_loop(0, 128, unroll=4)
                def _(l):
                    lcol = jnp.full((16,), l, dtype=jnp.int32)
                    for dq in range(4):
                        vals = rows[b][l, pl.ds(dq * 16, 16)]
                        plsc.store_scatter(
                            dst[b], [dvecs[dq], lcol], vals * SCALE
                        )

                @pl.when(s < S - 2)
                def _():
                    gather(s + 2, b).start()

                for c in out_copies(s, b):
                    c.start()

        for b in range(2):
            for c in out_copies(S - 2 + b, b):
                c.wait()

    o5 = emb(tp2, xt2)
    return o5.transpose(2, 4, 0, 1, 3).reshape(B, S, EMBED_DIM)
